# baseline (speedup 1.0000x reference)
import functools

import jax
import jax.numpy as jnp
from jax import lax
from jax.experimental import pallas as pl
from jax.experimental.pallas import tpu as pltpu
from jax.experimental.pallas import tpu_sc as plsc

_B, _T, _H = 1024, 20, 64
_NCAT = 4
_NC, _NS = 2, 16
_NW = _NC * _NS
_SROWS = _B * _NCAT
_TROWS = _B * _T * _NCAT
_S_PW = _SROWS // _NW
_T_PW = _TROWS // _NW
_TCH = 128
_TNCH = _T_PW // _TCH
_BB = 16


def _sc_gather(stat_tab, temp_tab, sidx, tidx):
    mesh = plsc.VectorSubcoreMesh(
        core_axis_name="c", subcore_axis_name="s",
        num_cores=_NC, num_subcores=_NS)

    @functools.partial(
        pl.kernel,
        out_type=[
            jax.ShapeDtypeStruct((_SROWS, _H), jnp.float32),
            jax.ShapeDtypeStruct((_TROWS, _H), jnp.float32),
        ],
        mesh=mesh,
        scratch_types=[
            pltpu.VMEM((_S_PW,), jnp.int32),
            pltpu.VMEM((_TNCH, _TCH), jnp.int32),
            pltpu.VMEM((_S_PW, _H), jnp.float32),
            pltpu.VMEM((_TCH, _H), jnp.float32),
            pltpu.VMEM((_TCH, _H), jnp.float32),
            pltpu.SemaphoreType.DMA,
            pltpu.SemaphoreType.DMA,
        ],
        compiler_params=pltpu.CompilerParams(use_tc_tiling_on_sc=False),
    )
    def k(stat_hbm, temp_hbm, sidx_hbm, tidx_hbm, gs_out, gt_out,
          sidx_v, tidx_v, srow_v, trow_a, trow_b, sem_a, sem_b):
        wid = lax.axis_index("s") * _NC + lax.axis_index("c")
        sbase = wid * _S_PW
        tbase = wid * _T_PW
        pltpu.sync_copy(sidx_hbm.at[wid], sidx_v)
        pltpu.sync_copy(tidx_hbm.at[wid], tidx_v)
        pltpu.async_copy(stat_hbm.at[sidx_v], srow_v, sem_a).wait()
        pltpu.sync_copy(srow_v, gs_out.at[pl.ds(sbase, _S_PW)])
        bufs = (trow_a, trow_b)
        sems = (sem_a, sem_b)
        descs = [None] * _TNCH
        descs[0] = pltpu.async_copy(temp_hbm.at[tidx_v.at[0]], trow_a, sem_a)
        for j in range(_TNCH):
            if j + 1 < _TNCH:
                descs[j + 1] = pltpu.async_copy(
                    temp_hbm.at[tidx_v.at[j + 1]],
                    bufs[(j + 1) % 2], sems[(j + 1) % 2])
            descs[j].wait()
            pltpu.sync_copy(bufs[j % 2], gt_out.at[pl.ds(tbase + j * _TCH, _TCH)])

    return k(stat_tab, temp_tab, sidx, tidx)


def _gelu_exact(x):
    ax = jnp.abs(x) * 0.7071067811865476
    t = 1.0 / (1.0 + 0.3275911 * ax)
    poly = t * (0.254829592 + t * (-0.284496736 + t * (
        1.421413741 + t * (-1.453152027 + t * 1.061405429))))
    erf_ax = 1.0 - poly * jnp.exp(-ax * ax)
    erf_x = jnp.where(x >= 0.0, erf_ax, -erf_ax)
    return 0.5 * x * (1.0 + erf_x)


def _fuse_body(gt, gs, me, st, tg, mv, mb, sv, sb, tv, tb, k_o, s_o, t_o):
    lane_t = lax.broadcasted_iota(jnp.int32, (_BB, _T, 11, 128), 3)
    mev = me[...]
    cont_t = jnp.where(lane_t < _H, mev[..., 0:1], mev[..., 1:2]) * mv[...] + mb[...]
    k_o[...] = jnp.concatenate([_gelu_exact(gt[...]), cont_t], axis=2)
    lane_s = lax.broadcasted_iota(jnp.int32, (_BB, 11, 128), 2)
    stv = st[...]
    cont_s = jnp.where(lane_s < _H, stv[..., 0:1], stv[..., 1:2]) * sv[...] + sb[...]
    s_o[...] = jnp.concatenate([_gelu_exact(gs[...]), cont_s], axis=1)
    lane_g = lax.broadcasted_iota(jnp.int32, (_BB, 10, 128), 2)
    tgv = tg[...]
    t_o[...] = jnp.where(lane_g < _H, tgv[..., 0:1], tgv[..., 1:2]) * tv[...] + tb[...]


def _tc_fuse(gt_pack, gs_pack, me_c2, st_c2, tg_c2, mv2, mb_pack, sv_pack,
             sb_pack, tv2, tb2):
    nblk = _B // _BB
    grid = (nblk,)
    return pl.pallas_call(
        _fuse_body,
        grid=grid,
        in_specs=[
            pl.BlockSpec((_BB, _T, 2, 128), lambda i: (i, 0, 0, 0)),
            pl.BlockSpec((_BB, 2, 128), lambda i: (i, 0, 0)),
            pl.BlockSpec((_BB, _T, 11, 2), lambda i: (i, 0, 0, 0)),
            pl.BlockSpec((_BB, 11, 2), lambda i: (i, 0, 0)),
            pl.BlockSpec((_BB, 10, 2), lambda i: (i, 0, 0)),
            pl.BlockSpec((1, 128), lambda i: (0, 0)),
            pl.BlockSpec((11, 128), lambda i: (0, 0)),
            pl.BlockSpec((11, 128), lambda i: (0, 0)),
            pl.BlockSpec((11, 128), lambda i: (0, 0)),
            pl.BlockSpec((1, 128), lambda i: (0, 0)),
            pl.BlockSpec((1, 128), lambda i: (0, 0)),
        ],
        out_specs=[
            pl.BlockSpec((_BB, _T, 13, 128), lambda i: (i, 0, 0, 0)),
            pl.BlockSpec((_BB, 13, 128), lambda i: (i, 0, 0)),
            pl.BlockSpec((_BB, 10, 128), lambda i: (i, 0, 0)),
        ],
        out_shape=[
            jax.ShapeDtypeStruct((_B, _T, 13, 128), jnp.float32),
            jax.ShapeDtypeStruct((_B, 13, 128), jnp.float32),
            jax.ShapeDtypeStruct((_B, 10, 128), jnp.float32),
        ],
        compiler_params=pltpu.CompilerParams(
            dimension_semantics=("arbitrary",)),
    )(gt_pack, gs_pack, me_c2, st_c2, tg_c2, mv2, mb_pack, sv_pack, sb_pack,
      tv2, tb2)


def kernel(target_inp, stat_exog, multi_exog, stat_emb_tables, temp_emb_tables,
           stat_vectors, stat_bias, multi_vectors, multi_bias, tgt_vectors,
           tgt_bias):
    V = stat_emb_tables.shape[1]
    stat_tab = stat_emb_tables.reshape(-1, _H)
    temp_tab = temp_emb_tables.reshape(-1, _H)
    offs = jnp.arange(_NCAT, dtype=jnp.int32) * V
    sidx = (stat_exog[:, :_NCAT] + offs).reshape(_NW, _S_PW)
    tidx = (multi_exog[:, :, :_NCAT].astype(jnp.int32) + offs).reshape(
        _NW, _TNCH, _TCH)
    gs, gt = _sc_gather(stat_tab, temp_tab, sidx, tidx)

    mvrow = multi_vectors[_NCAT]
    mv2 = jnp.concatenate([mvrow, mvrow]).reshape(1, 128)
    mb_pack = multi_bias[_NCAT:].reshape(11, 128)
    sv_pack = stat_vectors[_NCAT:].reshape(11, 128)
    sb_pack = stat_bias[_NCAT:].reshape(11, 128)
    tv2 = jnp.concatenate([tgt_vectors[0], tgt_vectors[0]]).reshape(1, 128)
    tb2 = jnp.concatenate([tgt_bias[0], tgt_bias[0]]).reshape(1, 128)

    k_pack, s_pack, t_pack = _tc_fuse(
        gt.reshape(_B, _T, 2, 128),
        gs.reshape(_B, 2, 128),
        multi_exog[:, :, _NCAT:].reshape(_B, _T, 11, 2),
        stat_exog[:, _NCAT:].astype(jnp.float32).reshape(_B, 11, 2),
        target_inp.reshape(_B, 10, 2),
        mv2, mb_pack, sv_pack, sb_pack, tv2, tb2)

    s_inp = s_pack.reshape(_B, 26, _H)
    k_inp = k_pack.reshape(_B, _T, 26, _H)
    tgt_out = t_pack.reshape(_B, _T, 1, _H)
    return (s_inp, k_inp, tgt_out)


# speedup vs baseline: 1.6664x; 1.6664x over previous
import functools

import jax
import jax.numpy as jnp
from jax import lax
from jax.experimental import pallas as pl
from jax.experimental.pallas import tpu as pltpu
from jax.experimental.pallas import tpu_sc as plsc

_B, _T, _H = 1024, 20, 64
_NCAT = 4
_NC, _NS = 2, 16
_NW = _NC * _NS
_SROWS = _B * _NCAT
_TROWS = _B * _T * _NCAT
_S_PW = _SROWS // _NW
_T_PW = _TROWS // _NW
_TCH = 128
_TNCH = _T_PW // _TCH
_BB = 16


def _sc_gather(stat_tab, temp_tab, sidx, tidx):
    mesh = plsc.VectorSubcoreMesh(
        core_axis_name="c", subcore_axis_name="s",
        num_cores=_NC, num_subcores=_NS)

    @functools.partial(
        pl.kernel,
        out_type=[
            jax.ShapeDtypeStruct((_SROWS, _H), jnp.float32),
            jax.ShapeDtypeStruct((_TROWS, _H), jnp.float32),
        ],
        mesh=mesh,
        scratch_types=[
            pltpu.VMEM((_S_PW,), jnp.int32),
            pltpu.VMEM((_TNCH, _TCH), jnp.int32),
            pltpu.VMEM((_S_PW, _H), jnp.float32),
            pltpu.VMEM((_TCH, _H), jnp.float32),
            pltpu.VMEM((_TCH, _H), jnp.float32),
            pltpu.SemaphoreType.DMA,
            pltpu.SemaphoreType.DMA,
        ],
        compiler_params=pltpu.CompilerParams(use_tc_tiling_on_sc=False),
    )
    def k(stat_hbm, temp_hbm, sidx_hbm, tidx_hbm, gs_out, gt_out,
          sidx_v, tidx_v, srow_v, trow_a, trow_b, sem_a, sem_b):
        wid = lax.axis_index("s") * _NC + lax.axis_index("c")
        sbase = wid * _S_PW
        tbase = wid * _T_PW
        pltpu.sync_copy(sidx_hbm.at[wid], sidx_v)
        pltpu.sync_copy(tidx_hbm.at[wid], tidx_v)
        pltpu.async_copy(stat_hbm.at[sidx_v], srow_v, sem_a).wait()
        pltpu.sync_copy(srow_v, gs_out.at[pl.ds(sbase, _S_PW)])
        bufs = (trow_a, trow_b)
        sems = (sem_a, sem_b)
        descs = [None] * _TNCH
        descs[0] = pltpu.async_copy(temp_hbm.at[tidx_v.at[0]], trow_a, sem_a)
        for j in range(_TNCH):
            if j + 1 < _TNCH:
                descs[j + 1] = pltpu.async_copy(
                    temp_hbm.at[tidx_v.at[j + 1]],
                    bufs[(j + 1) % 2], sems[(j + 1) % 2])
            descs[j].wait()
            pltpu.sync_copy(bufs[j % 2], gt_out.at[pl.ds(tbase + j * _TCH, _TCH)])

    return k(stat_tab, temp_tab, sidx, tidx)


_BL = 256


def _gelu_exact(x):
    ax = jnp.abs(x) * 0.7071067811865476
    t = 1.0 / (1.0 + 0.3275911 * ax)
    poly = t * (0.254829592 + t * (-0.284496736 + t * (
        1.421413741 + t * (-1.453152027 + t * 1.061405429))))
    erf_ax = 1.0 - poly * jnp.exp(-ax * ax)
    erf_x = jnp.where(x >= 0.0, erf_ax, -erf_ax)
    return 0.5 * x * (1.0 + erf_x)


def _fuse_body(gt, gs, me, st, tg, mv, mb, sv, sb, tv, tb, k_o, s_o, t_o):
    t_idx = pl.program_id(1)
    gtt = jnp.transpose(gt[:, 0, 0, :], (1, 0))
    for c in range(4):
        k_o[0, c] = _gelu_exact(gtt[c * _H:(c + 1) * _H, :])
    mvc = mv[...]
    for c in range(4, 26):
        k_o[0, c] = me[pl.ds(c, 1), 0, 0, :] * mvc + mb[:, pl.ds(c, 1)]
    @pl.when(t_idx == 0)
    def _static():
        gst = jnp.transpose(gs[...], (1, 0))
        for c in range(4):
            s_o[c] = _gelu_exact(gst[c * _H:(c + 1) * _H, :])
        for c in range(4, 26):
            s_o[c] = st[pl.ds(c, 1), :] * sv[:, pl.ds(c, 1)] + sb[:, pl.ds(c, 1)]
    t_o[0] = tg[:, 0, :] * tv[...] + tb[...]


def _tc_fuse(gt_r, gs_r, me_t, st_t, tg_t, mv_col, mbT, svT, sbT, tv_col,
             tb_col):
    nbc = _B // _BL
    return pl.pallas_call(
        _fuse_body,
        grid=(nbc, _T),
        in_specs=[
            pl.BlockSpec((_BL, 1, 1, 256), lambda i, t: (i, t, 0, 0)),
            pl.BlockSpec((_BL, 256), lambda i, t: (i, 0)),
            pl.BlockSpec((26, 1, 1, _BL), lambda i, t: (0, t, 0, i)),
            pl.BlockSpec((26, _BL), lambda i, t: (0, i)),
            pl.BlockSpec((1, 1, _BL), lambda i, t: (t, 0, i)),
            pl.BlockSpec((_H, 1), lambda i, t: (0, 0)),
            pl.BlockSpec((_H, 26), lambda i, t: (0, 0)),
            pl.BlockSpec((_H, 26), lambda i, t: (0, 0)),
            pl.BlockSpec((_H, 26), lambda i, t: (0, 0)),
            pl.BlockSpec((_H, 1), lambda i, t: (0, 0)),
            pl.BlockSpec((_H, 1), lambda i, t: (0, 0)),
        ],
        out_specs=[
            pl.BlockSpec((1, 26, _H, _BL), lambda i, t: (t, 0, 0, i)),
            pl.BlockSpec((26, _H, _BL), lambda i, t: (0, 0, i)),
            pl.BlockSpec((1, _H, _BL), lambda i, t: (t, 0, i)),
        ],
        out_shape=[
            jax.ShapeDtypeStruct((_T, 26, _H, _B), jnp.float32),
            jax.ShapeDtypeStruct((26, _H, _B), jnp.float32),
            jax.ShapeDtypeStruct((_T, _H, _B), jnp.float32),
        ],
        compiler_params=pltpu.CompilerParams(
            dimension_semantics=("arbitrary", "arbitrary")),
    )(gt_r, gs_r, me_t, st_t, tg_t, mv_col, mbT, svT, sbT, tv_col, tb_col)


def kernel(target_inp, stat_exog, multi_exog, stat_emb_tables, temp_emb_tables,
           stat_vectors, stat_bias, multi_vectors, multi_bias, tgt_vectors,
           tgt_bias):
    V = stat_emb_tables.shape[1]
    stat_tab = stat_emb_tables.reshape(-1, _H)
    temp_tab = temp_emb_tables.reshape(-1, _H)
    offs = jnp.arange(_NCAT, dtype=jnp.int32) * V
    sidx = (stat_exog[:, :_NCAT] + offs).reshape(_NW, _S_PW)
    tidx = (multi_exog[:, :, :_NCAT].astype(jnp.int32) + offs).reshape(
        _NW, _TNCH, _TCH)
    gs, gt = _sc_gather(stat_tab, temp_tab, sidx, tidx)

    me_t = jnp.transpose(multi_exog, (2, 1, 0))
    st_t = jnp.transpose(stat_exog, (1, 0)).astype(jnp.float32)
    tg_t = jnp.transpose(target_inp, (1, 2, 0)).reshape(_T, _B)
    mv_col = multi_vectors[_NCAT].reshape(_H, 1)
    mbT = multi_bias.T
    svT = stat_vectors.T
    sbT = stat_bias.T
    tv_col = tgt_vectors.reshape(_H, 1)
    tb_col = tgt_bias.reshape(_H, 1)

    k_t, s_t, t_t = _tc_fuse(
        gt.reshape(_B, _T, 1, _NCAT * _H),
        gs.reshape(_B, _NCAT * _H),
        me_t.reshape(26, _T, 1, _B), st_t, tg_t.reshape(_T, 1, _B),
        mv_col, mbT, svT, sbT, tv_col, tb_col)

    s_inp = jnp.transpose(s_t, (2, 0, 1))
    k_inp = jnp.transpose(k_t, (3, 0, 1, 2))
    tgt_out = jnp.transpose(t_t.reshape(_T, 1, _H, _B), (3, 0, 1, 2))
    return (s_inp, k_inp, tgt_out)


# speedup vs baseline: 1.7297x; 1.0380x over previous
import functools

import jax
import jax.numpy as jnp
from jax import lax
from jax.experimental import pallas as pl
from jax.experimental.pallas import tpu as pltpu
from jax.experimental.pallas import tpu_sc as plsc

_B, _T, _H = 1024, 20, 64
_NCAT = 4
_NC, _NS = 2, 16
_NW = _NC * _NS
_SROWS = _B * _NCAT
_TROWS = _B * _T * _NCAT
_S_PW = _SROWS // _NW
_T_PW = _TROWS // _NW
_TCH = 128
_TNCH = _T_PW // _TCH
_BB = 16


def _sc_gather(stat_tab, temp_tab, sidx, tidx):
    mesh = plsc.VectorSubcoreMesh(
        core_axis_name="c", subcore_axis_name="s",
        num_cores=_NC, num_subcores=_NS)

    @functools.partial(
        pl.kernel,
        out_type=[
            jax.ShapeDtypeStruct((_SROWS, _H), jnp.float32),
            jax.ShapeDtypeStruct((_TROWS, _H), jnp.float32),
        ],
        mesh=mesh,
        scratch_types=[
            pltpu.VMEM((_S_PW,), jnp.int32),
            pltpu.VMEM((_TNCH, _TCH), jnp.int32),
            pltpu.VMEM((_S_PW, _H), jnp.float32),
            pltpu.VMEM((_TCH, _H), jnp.float32),
            pltpu.VMEM((_TCH, _H), jnp.float32),
            pltpu.SemaphoreType.DMA,
            pltpu.SemaphoreType.DMA,
        ],
        compiler_params=pltpu.CompilerParams(use_tc_tiling_on_sc=False),
    )
    def k(stat_hbm, temp_hbm, sidx_hbm, tidx_hbm, gs_out, gt_out,
          sidx_v, tidx_v, srow_v, trow_a, trow_b, sem_a, sem_b):
        wid = lax.axis_index("s") * _NC + lax.axis_index("c")
        sbase = wid * _S_PW
        tbase = wid * _T_PW
        pltpu.sync_copy(sidx_hbm.at[wid], sidx_v)
        pltpu.sync_copy(tidx_hbm.at[wid], tidx_v)
        pltpu.async_copy(stat_hbm.at[sidx_v], srow_v, sem_a).wait()
        pltpu.sync_copy(srow_v, gs_out.at[pl.ds(sbase, _S_PW)])
        bufs = (trow_a, trow_b)
        sems = (sem_a, sem_b)
        descs = [None] * _TNCH
        descs[0] = pltpu.async_copy(temp_hbm.at[tidx_v.at[0]], trow_a, sem_a)
        for j in range(_TNCH):
            if j + 1 < _TNCH:
                descs[j + 1] = pltpu.async_copy(
                    temp_hbm.at[tidx_v.at[j + 1]],
                    bufs[(j + 1) % 2], sems[(j + 1) % 2])
            descs[j].wait()
            pltpu.sync_copy(bufs[j % 2], gt_out.at[pl.ds(tbase + j * _TCH, _TCH)])

    return k(stat_tab, temp_tab, sidx, tidx)


_BL = 512


def _gelu_exact(x):
    ax = jnp.abs(x) * 0.7071067811865476
    t = 1.0 / (1.0 + 0.3275911 * ax)
    poly = t * (0.254829592 + t * (-0.284496736 + t * (
        1.421413741 + t * (-1.453152027 + t * 1.061405429))))
    erf_ax = 1.0 - poly * jnp.exp(-ax * ax)
    erf_x = jnp.where(x >= 0.0, erf_ax, -erf_ax)
    return 0.5 * x * (1.0 + erf_x)


def _fuse_body(gt, gs, me, st, tg, kW, kB, sW, sB, tv, tb, k_o, s_o, t_o):
    t_idx = pl.program_id(1)
    gtt = jnp.transpose(gt[:, 0, 0, :], (1, 0))
    for c in range(4):
        k_o[0, c] = _gelu_exact(gtt[c * _H:(c + 1) * _H, :])
    mm = lax.dot_general(kW[...], me[:, 0, 0, :], (((1,), (0,)), ((), ())),
                         preferred_element_type=jnp.float32) + kB[...]
    for j in range(22):
        k_o[0, 4 + j] = mm[j * _H:(j + 1) * _H, :]
    @pl.when(t_idx == 0)
    def _static():
        gst = jnp.transpose(gs[...], (1, 0))
        for c in range(4):
            s_o[c] = _gelu_exact(gst[c * _H:(c + 1) * _H, :])
        sm = lax.dot_general(sW[...], st[...], (((1,), (0,)), ((), ())),
                             preferred_element_type=jnp.float32) + sB[...]
        for j in range(22):
            s_o[4 + j] = sm[j * _H:(j + 1) * _H, :]
    t_o[0] = tg[:, 0, :] * tv[...] + tb[...]


def _tc_fuse(gt_r, gs_r, me_t, st_t, tg_t, kW, kB, sW, sB, tv_col, tb_col):
    nbc = _B // _BL
    return pl.pallas_call(
        _fuse_body,
        grid=(nbc, _T),
        in_specs=[
            pl.BlockSpec((_BL, 1, 1, 256), lambda i, t: (i, t, 0, 0)),
            pl.BlockSpec((_BL, 256), lambda i, t: (i, 0)),
            pl.BlockSpec((26, 1, 1, _BL), lambda i, t: (0, t, 0, i)),
            pl.BlockSpec((26, _BL), lambda i, t: (0, i)),
            pl.BlockSpec((1, 1, _BL), lambda i, t: (t, 0, i)),
            pl.BlockSpec((1408, 26), lambda i, t: (0, 0)),
            pl.BlockSpec((1408, 1), lambda i, t: (0, 0)),
            pl.BlockSpec((1408, 26), lambda i, t: (0, 0)),
            pl.BlockSpec((1408, 1), lambda i, t: (0, 0)),
            pl.BlockSpec((_H, 1), lambda i, t: (0, 0)),
            pl.BlockSpec((_H, 1), lambda i, t: (0, 0)),
        ],
        out_specs=[
            pl.BlockSpec((1, 26, _H, _BL), lambda i, t: (t, 0, 0, i)),
            pl.BlockSpec((26, _H, _BL), lambda i, t: (0, 0, i)),
            pl.BlockSpec((1, _H, _BL), lambda i, t: (t, 0, i)),
        ],
        out_shape=[
            jax.ShapeDtypeStruct((_T, 26, _H, _B), jnp.float32),
            jax.ShapeDtypeStruct((26, _H, _B), jnp.float32),
            jax.ShapeDtypeStruct((_T, _H, _B), jnp.float32),
        ],
        compiler_params=pltpu.CompilerParams(
            dimension_semantics=("arbitrary", "arbitrary")),
    )(gt_r, gs_r, me_t, st_t, tg_t, kW, kB, sW, sB, tv_col, tb_col)


def kernel(target_inp, stat_exog, multi_exog, stat_emb_tables, temp_emb_tables,
           stat_vectors, stat_bias, multi_vectors, multi_bias, tgt_vectors,
           tgt_bias):
    V = stat_emb_tables.shape[1]
    stat_tab = stat_emb_tables.reshape(-1, _H)
    temp_tab = temp_emb_tables.reshape(-1, _H)
    offs = jnp.arange(_NCAT, dtype=jnp.int32) * V
    sidx = (stat_exog[:, :_NCAT] + offs).reshape(_NW, _S_PW)
    tidx = (multi_exog[:, :, :_NCAT].astype(jnp.int32) + offs).reshape(
        _NW, _TNCH, _TCH)
    gs, gt = _sc_gather(stat_tab, temp_tab, sidx, tidx)

    me_t = jnp.transpose(multi_exog, (2, 1, 0))
    st_t = jnp.transpose(stat_exog, (1, 0)).astype(jnp.float32)
    tg_t = jnp.transpose(target_inp, (1, 2, 0)).reshape(_T, _B)
    eye22 = jnp.eye(22, dtype=jnp.float32)
    kW = jnp.concatenate([
        jnp.zeros((1408, _NCAT), jnp.float32),
        (eye22[:, None, :] * multi_vectors[_NCAT][None, :, None]
         ).reshape(1408, 22)], axis=1)
    kB = multi_bias[_NCAT:].reshape(1408, 1)
    sW = jnp.concatenate([
        jnp.zeros((1408, _NCAT), jnp.float32),
        (eye22[:, None, :] * jnp.transpose(stat_vectors[_NCAT:], (0, 1))[:, :, None]
         ).reshape(1408, 22)], axis=1)
    sB = stat_bias[_NCAT:].reshape(1408, 1)
    tv_col = tgt_vectors.reshape(_H, 1)
    tb_col = tgt_bias.reshape(_H, 1)

    k_t, s_t, t_t = _tc_fuse(
        gt.reshape(_B, _T, 1, _NCAT * _H),
        gs.reshape(_B, _NCAT * _H),
        me_t.reshape(26, _T, 1, _B), st_t, tg_t.reshape(_T, 1, _B),
        kW, kB, sW, sB, tv_col, tb_col)

    s_inp = jnp.transpose(s_t, (2, 0, 1))
    k_inp = jnp.transpose(k_t, (3, 0, 1, 2))
    tgt_out = jnp.transpose(t_t.reshape(_T, 1, _H, _B), (3, 0, 1, 2))
    return (s_inp, k_inp, tgt_out)
